# baseline (speedup 1.0000x reference)
import jax
import jax.numpy as jnp
from jax.experimental import pallas as pl
from jax.experimental.pallas import tpu as pltpu


def _body(cid_ref, f_ref, p_ref, c_ref, po_ref, co_ref, *, blk, n_add):
    i = pl.program_id(0)
    po_ref[...] = p_ref[...]
    co_ref[...] = c_ref[...]
    cid = cid_ref[0]
    base = i * blk

    @pl.when(jnp.logical_and(cid >= base, cid < base + blk))
    def _():
        f = f_ref[...]
        nrm = jnp.sqrt(jnp.sum(f * f, axis=1, keepdims=True))
        fn = f / jnp.maximum(nrm, 1e-12)
        r = cid - base
        po_ref[pl.ds(r, 1), :, :] = fn[None]
        co_ref[pl.ds(r, 1), :] = jnp.ones((1, c_ref.shape[1]), jnp.int32)


def kernel(features, prototypes, counts, class_id):
    C, P, D = prototypes.shape
    n_add = min(features.shape[0], P)
    cid = jnp.asarray(class_id, jnp.int32).reshape((1,))
    feats = features[:n_add]

    BLK = 8
    assert C % BLK == 0
    grid = (C // BLK,)

    import functools
    body = functools.partial(_body, blk=BLK, n_add=n_add)

    grid_spec = pltpu.PrefetchScalarGridSpec(
        num_scalar_prefetch=1,
        grid=grid,
        in_specs=[
            pl.BlockSpec((n_add, D), lambda i, cid_ref: (0, 0)),
            pl.BlockSpec((BLK, P, D), lambda i, cid_ref: (i, 0, 0)),
            pl.BlockSpec((BLK, P), lambda i, cid_ref: (i, 0)),
        ],
        out_specs=[
            pl.BlockSpec((BLK, P, D), lambda i, cid_ref: (i, 0, 0)),
            pl.BlockSpec((BLK, P), lambda i, cid_ref: (i, 0)),
        ],
    )
    protos_out, counts_out = pl.pallas_call(
        body,
        grid_spec=grid_spec,
        out_shape=[
            jax.ShapeDtypeStruct((C, P, D), jnp.float32),
            jax.ShapeDtypeStruct((C, P), jnp.int32),
        ],
    )(cid, feats, prototypes, counts)
    return protos_out, counts_out
